# baseline (speedup 1.0000x reference)
import jax
import jax.numpy as jnp
from jax import lax
from jax.experimental import pallas as pl
from jax.experimental.pallas import tpu as pltpu
from jax.experimental.pallas import tpu_sc as plsc

D = 128
E = 320000
NC = 2
NS = 16
NW = NC * NS
EPW = E // NW
C = 80
NCHUNK = EPW // C
NBUF = 4
UNROLL = 8
W = D // 2


def _edge_dot_body(src_hbm, tgt_hbm, sid_hbm, tid_hbm, out_hbm,
                   sidx_v, tidx_v, out_v, *bufs_and_sems):
    srows = bufs_and_sems[0:NBUF]
    trows = bufs_and_sems[NBUF:2 * NBUF]
    sems = bufs_and_sems[2 * NBUF:]
    sub = lax.axis_index("s")
    wid = sub * NC + lax.axis_index("c")
    wbase = wid * EPW

    pltpu.sync_copy(sid_hbm.at[pl.ds(wbase, EPW)], sidx_v)
    pltpu.sync_copy(tid_hbm.at[pl.ds(wbase, EPW)], tidx_v)

    def fire(ci, b):
        pltpu.async_copy(
            src_hbm.at[sidx_v.at[pl.ds(ci * C, C)]], srows[b], sems[2 * b])
        pltpu.async_copy(
            tgt_hbm.at[tidx_v.at[pl.ds(ci * C, C)]], trows[b], sems[2 * b + 1])

    def wait(b):
        pltpu.make_async_copy(
            src_hbm.at[pl.ds(0, C)], srows[b], sems[2 * b]).wait()
        pltpu.make_async_copy(
            tgt_hbm.at[pl.ds(0, C)], trows[b], sems[2 * b + 1]).wait()

    def compute(ci, b):
        sb = srows[b]
        tb = trows[b]
        lane = lax.iota(jnp.int32, 16)
        for g in range(C // 16):
            rows = lane + g * 16
            zero = jnp.zeros((16,), jnp.float32)

            def d_blk(k, carry):
                acc0, acc1 = carry
                base = k * UNROLL
                for j in range(UNROLL):
                    col = (jnp.full((16,), base + j, jnp.int32) + lane) & (W - 1)
                    s = plsc.load_gather(sb, [rows, col])
                    t = plsc.load_gather(tb, [rows, col])
                    p = plsc.bitcast(s, jnp.bfloat16) * plsc.bitcast(t, jnp.bfloat16)
                    pe, po = plsc.unpack(p, format=plsc.PackFormat.INTERLEAVED)
                    acc0 = acc0 + pe
                    acc1 = acc1 + po
                return acc0, acc1

            acc0, acc1 = lax.fori_loop(0, W // UNROLL, d_blk, (zero, zero))
            out_v[pl.ds(ci * C + g * 16, 16)] = acc0 + acc1

    for b in range(NBUF):
        fire(b, b)

    def loop_body(i, carry):
        for b in range(NBUF):
            ci = i * NBUF + b

            @pl.when(ci < NCHUNK)
            def _():
                wait(b)
                compute(ci, b)

                @pl.when(ci + NBUF < NCHUNK)
                def _():
                    fire(ci + NBUF, b)

        return carry

    lax.fori_loop(0, (NCHUNK + NBUF - 1) // NBUF, loop_body, 0)
    pltpu.sync_copy(out_v, out_hbm.at[pl.ds(wbase, EPW)])


def kernel(node_src_feats, node_tgt_feats, edge_ids):
    eids = edge_ids.astype(jnp.int32)
    sids = eids[0]
    tids = eids[1]
    nn = node_src_feats.shape[0]
    src_w = lax.bitcast_convert_type(
        node_src_feats.astype(jnp.bfloat16).reshape(nn, W, 2), jnp.int32)
    tgt_w = lax.bitcast_convert_type(
        node_tgt_feats.astype(jnp.bfloat16).reshape(nn, W, 2), jnp.int32)
    mesh = plsc.VectorSubcoreMesh(core_axis_name="c", subcore_axis_name="s")
    fn = pl.kernel(
        _edge_dot_body,
        out_type=jax.ShapeDtypeStruct((E,), jnp.float32),
        mesh=mesh,
        scratch_types=[
            pltpu.VMEM((EPW,), jnp.int32),
            pltpu.VMEM((EPW,), jnp.int32),
            pltpu.VMEM((EPW,), jnp.float32),
        ] + [pltpu.VMEM((C, W), jnp.int32) for _ in range(2 * NBUF)]
          + [pltpu.SemaphoreType.DMA for _ in range(2 * NBUF)],
        compiler_params=pltpu.CompilerParams(
            needs_layout_passes=False, use_tc_tiling_on_sc=False),
    )
    order = jnp.argsort(sids)
    res = fn(src_w, tgt_w, sids[order], tids[order])
    return jnp.zeros((E,), jnp.float32).at[order].set(res)


# speedup vs baseline: 9.0131x; 9.0131x over previous
import jax
import jax.numpy as jnp
from jax import lax
from jax.experimental import pallas as pl
from jax.experimental.pallas import tpu as pltpu
from jax.experimental.pallas import tpu_sc as plsc

D = 128
E = 320000
NC = 2
NS = 16
NW = NC * NS
EPW = E // NW
C = 400
NCHUNK = EPW // C
NBUF = 2
UNROLL = 8
W = D // 2


def _edge_dot_body(src_hbm, tgt_hbm, sid_hbm, tid_hbm, out_hbm,
                   sidx_v, tidx_v, *bufs_and_sems):
    srows = bufs_and_sems[0:NBUF]
    trows = bufs_and_sems[NBUF:2 * NBUF]
    outs = bufs_and_sems[2 * NBUF:3 * NBUF]
    sems = bufs_and_sems[3 * NBUF:]
    wid = lax.axis_index("s") * NC + lax.axis_index("c")
    wbase = wid * EPW

    pltpu.sync_copy(sid_hbm.at[pl.ds(wbase, EPW)], sidx_v)
    pltpu.sync_copy(tid_hbm.at[pl.ds(wbase, EPW)], tidx_v)

    def fire(ci, b):
        pltpu.async_copy(
            src_hbm.at[sidx_v.at[pl.ds(ci * C, C)]], srows[b], sems[3 * b])
        pltpu.async_copy(
            tgt_hbm.at[tidx_v.at[pl.ds(ci * C, C)]], trows[b], sems[3 * b + 1])

    def wait(b):
        pltpu.make_async_copy(
            src_hbm.at[pl.ds(0, C)], srows[b], sems[3 * b]).wait()
        pltpu.make_async_copy(
            tgt_hbm.at[pl.ds(0, C)], trows[b], sems[3 * b + 1]).wait()

    def wait_out(b):
        pltpu.make_async_copy(
            outs[b], out_hbm.at[pl.ds(0, C)], sems[3 * b + 2]).wait()

    def compute(ci, b):
        sb = srows[b]
        tb = trows[b]
        lane = lax.iota(jnp.int32, 16)
        for g in range(C // 16):
            rows = lane + g * 16
            zero = jnp.zeros((16,), jnp.float32)

            def d_blk(k, carry):
                acc0, acc1 = carry
                base = k * UNROLL
                for j in range(UNROLL):
                    col = (jnp.full((16,), base + j, jnp.int32) + lane) & (W - 1)
                    s = plsc.load_gather(sb, [rows, col])
                    t = plsc.load_gather(tb, [rows, col])
                    p = plsc.bitcast(s, jnp.bfloat16) * plsc.bitcast(t, jnp.bfloat16)
                    pe, po = plsc.unpack(p, format=plsc.PackFormat.INTERLEAVED)
                    acc0 = acc0 + pe
                    acc1 = acc1 + po
                return acc0, acc1

            acc0, acc1 = lax.fori_loop(0, W // UNROLL, d_blk, (zero, zero))
            outs[b][pl.ds(g * 16, 16)] = acc0 + acc1
        pltpu.async_copy(
            outs[b], out_hbm.at[pl.ds(wbase + ci * C, C)], sems[3 * b + 2])

    for b in range(NBUF):
        fire(b, b)

    def loop_body(i, carry):
        for b in range(NBUF):
            ci = i * NBUF + b

            @pl.when(ci < NCHUNK)
            def _():
                wait(b)

                @pl.when(ci >= NBUF)
                def _():
                    wait_out(b)

                compute(ci, b)

                @pl.when(ci + NBUF < NCHUNK)
                def _():
                    fire(ci + NBUF, b)

        return carry

    lax.fori_loop(0, (NCHUNK + NBUF - 1) // NBUF, loop_body, 0)
    for b in range(NBUF):
        wait_out(b)


def kernel(node_src_feats, node_tgt_feats, edge_ids):
    eids = edge_ids.astype(jnp.int32)
    sids = eids[0]
    tids = eids[1]
    nn = node_src_feats.shape[0]
    src_w = lax.bitcast_convert_type(
        node_src_feats.astype(jnp.bfloat16).reshape(nn, W, 2), jnp.int32)
    tgt_w = lax.bitcast_convert_type(
        node_tgt_feats.astype(jnp.bfloat16).reshape(nn, W, 2), jnp.int32)
    mesh = plsc.VectorSubcoreMesh(core_axis_name="c", subcore_axis_name="s")
    fn = pl.kernel(
        _edge_dot_body,
        out_type=jax.ShapeDtypeStruct((E,), jnp.float32),
        mesh=mesh,
        scratch_types=[
            pltpu.VMEM((EPW,), jnp.int32),
            pltpu.VMEM((EPW,), jnp.int32),
        ] + [pltpu.VMEM((C, W), jnp.int32) for _ in range(2 * NBUF)]
          + [pltpu.VMEM((C,), jnp.float32) for _ in range(NBUF)]
          + [pltpu.SemaphoreType.DMA for _ in range(3 * NBUF)],
        compiler_params=pltpu.CompilerParams(
            needs_layout_passes=False, use_tc_tiling_on_sc=False),
    )
    return fn(src_w, tgt_w, sids, tids)


# speedup vs baseline: 9.0735x; 1.0067x over previous
import jax
import jax.numpy as jnp
from jax import lax
from jax.experimental import pallas as pl
from jax.experimental.pallas import tpu as pltpu
from jax.experimental.pallas import tpu_sc as plsc

D = 128
E = 320000
NC = 2
NS = 16
NW = NC * NS
EPW = E // NW
C = 80
NCHUNK = EPW // C
NBUF = 4
UNROLL = 8
W = D // 2


def _edge_dot_body(src_hbm, tgt_hbm, sid_hbm, tid_hbm, out_hbm,
                   sidx_v, tidx_v, *bufs_and_sems):
    srows = bufs_and_sems[0:NBUF]
    trows = bufs_and_sems[NBUF:2 * NBUF]
    outs = bufs_and_sems[2 * NBUF:3 * NBUF]
    sems = bufs_and_sems[3 * NBUF:]
    wid = lax.axis_index("s") * NC + lax.axis_index("c")
    wbase = wid * EPW

    pltpu.sync_copy(sid_hbm.at[pl.ds(wbase, EPW)], sidx_v)
    pltpu.sync_copy(tid_hbm.at[pl.ds(wbase, EPW)], tidx_v)

    def fire(ci, b):
        for k in range(C // 16):
            sidx = sidx_v[pl.ds(ci * C + k * 16, 16)]
            tidx = tidx_v[pl.ds(ci * C + k * 16, 16)]
            pltpu.async_copy(
                src_hbm.at[sidx], srows[b].at[pl.ds(k * 16, 16)], sems[3 * b])
            pltpu.async_copy(
                tgt_hbm.at[tidx], trows[b].at[pl.ds(k * 16, 16)], sems[3 * b + 1])

    def wait(b):
        pltpu.make_async_copy(
            src_hbm.at[pl.ds(0, C)], srows[b], sems[3 * b]).wait()
        pltpu.make_async_copy(
            tgt_hbm.at[pl.ds(0, C)], trows[b], sems[3 * b + 1]).wait()

    def wait_out(b):
        pltpu.make_async_copy(
            outs[b], out_hbm.at[pl.ds(0, C)], sems[3 * b + 2]).wait()

    def compute(ci, b):
        sb = srows[b]
        tb = trows[b]
        lane = lax.iota(jnp.int32, 16)
        for g in range(C // 16):
            rows = lane + g * 16
            zero = jnp.zeros((16,), jnp.float32)

            def d_blk(k, carry):
                acc0, acc1 = carry
                base = k * UNROLL
                for j in range(UNROLL):
                    col = (jnp.full((16,), base + j, jnp.int32) + lane) & (W - 1)
                    s = plsc.load_gather(sb, [rows, col])
                    t = plsc.load_gather(tb, [rows, col])
                    p = plsc.bitcast(s, jnp.bfloat16) * plsc.bitcast(t, jnp.bfloat16)
                    pe, po = plsc.unpack(p, format=plsc.PackFormat.INTERLEAVED)
                    acc0 = acc0 + pe
                    acc1 = acc1 + po
                return acc0, acc1

            acc0, acc1 = lax.fori_loop(0, W // UNROLL, d_blk, (zero, zero))
            outs[b][pl.ds(g * 16, 16)] = acc0 + acc1
        pltpu.async_copy(
            outs[b], out_hbm.at[pl.ds(wbase + ci * C, C)], sems[3 * b + 2])

    for b in range(NBUF):
        fire(b, b)

    def loop_body(i, carry):
        for b in range(NBUF):
            ci = i * NBUF + b

            @pl.when(ci < NCHUNK)
            def _():
                wait(b)

                @pl.when(ci >= NBUF)
                def _():
                    wait_out(b)

                compute(ci, b)

                @pl.when(ci + NBUF < NCHUNK)
                def _():
                    fire(ci + NBUF, b)

        return carry

    lax.fori_loop(0, (NCHUNK + NBUF - 1) // NBUF, loop_body, 0)
    for b in range(NBUF):
        wait_out(b)


def kernel(node_src_feats, node_tgt_feats, edge_ids):
    eids = edge_ids.astype(jnp.int32)
    sids = eids[0]
    tids = eids[1]
    nn = node_src_feats.shape[0]
    src_w = lax.bitcast_convert_type(
        node_src_feats.astype(jnp.bfloat16).reshape(nn, W, 2), jnp.int32)
    tgt_w = lax.bitcast_convert_type(
        node_tgt_feats.astype(jnp.bfloat16).reshape(nn, W, 2), jnp.int32)
    mesh = plsc.VectorSubcoreMesh(core_axis_name="c", subcore_axis_name="s")
    fn = pl.kernel(
        _edge_dot_body,
        out_type=jax.ShapeDtypeStruct((E,), jnp.float32),
        mesh=mesh,
        scratch_types=[
            pltpu.VMEM((EPW,), jnp.int32),
            pltpu.VMEM((EPW,), jnp.int32),
        ] + [pltpu.VMEM((C, W), jnp.int32) for _ in range(2 * NBUF)]
          + [pltpu.VMEM((C,), jnp.float32) for _ in range(NBUF)]
          + [pltpu.SemaphoreType.DMA for _ in range(3 * NBUF)],
        compiler_params=pltpu.CompilerParams(
            needs_layout_passes=False, use_tc_tiling_on_sc=False),
    )
    return fn(src_w, tgt_w, sids, tids)


# speedup vs baseline: 9.0911x; 1.0019x over previous
import jax
import jax.numpy as jnp
from jax import lax
from jax.experimental import pallas as pl
from jax.experimental.pallas import tpu as pltpu
from jax.experimental.pallas import tpu_sc as plsc

D = 128
E = 320000
NC = 2
NS = 16
NW = NC * NS
EPW = E // NW
C = 80
NCHUNK = EPW // C
NBUF = 8
UNROLL = 8
W = D // 2


def _edge_dot_body(src_hbm, tgt_hbm, sid_hbm, tid_hbm, out_hbm,
                   sidx_v, tidx_v, *bufs_and_sems):
    srows = bufs_and_sems[0:NBUF]
    trows = bufs_and_sems[NBUF:2 * NBUF]
    outs = bufs_and_sems[2 * NBUF:3 * NBUF]
    sems = bufs_and_sems[3 * NBUF:]
    wid = lax.axis_index("s") * NC + lax.axis_index("c")
    wbase = wid * EPW

    pltpu.sync_copy(sid_hbm.at[pl.ds(wbase, EPW)], sidx_v)
    pltpu.sync_copy(tid_hbm.at[pl.ds(wbase, EPW)], tidx_v)

    def fire(ci, b):
        pltpu.async_copy(
            src_hbm.at[sidx_v.at[pl.ds(ci * C, C)]], srows[b], sems[3 * b])
        pltpu.async_copy(
            tgt_hbm.at[tidx_v.at[pl.ds(ci * C, C)]], trows[b], sems[3 * b + 1])

    def wait(b):
        pltpu.make_async_copy(
            src_hbm.at[pl.ds(0, C)], srows[b], sems[3 * b]).wait()
        pltpu.make_async_copy(
            tgt_hbm.at[pl.ds(0, C)], trows[b], sems[3 * b + 1]).wait()

    def wait_out(b):
        pltpu.make_async_copy(
            outs[b], out_hbm.at[pl.ds(0, C)], sems[3 * b + 2]).wait()

    def compute(ci, b):
        sb = srows[b]
        tb = trows[b]
        lane = lax.iota(jnp.int32, 16)
        for g in range(C // 16):
            rows = lane + g * 16
            zero = jnp.zeros((16,), jnp.float32)

            def d_blk(k, carry):
                acc0, acc1 = carry
                base = k * UNROLL
                for j in range(UNROLL):
                    col = (jnp.full((16,), base + j, jnp.int32) + lane) & (W - 1)
                    s = plsc.load_gather(sb, [rows, col])
                    t = plsc.load_gather(tb, [rows, col])
                    p = plsc.bitcast(s, jnp.bfloat16) * plsc.bitcast(t, jnp.bfloat16)
                    pe, po = plsc.unpack(p, format=plsc.PackFormat.INTERLEAVED)
                    acc0 = acc0 + pe
                    acc1 = acc1 + po
                return acc0, acc1

            acc0, acc1 = lax.fori_loop(0, W // UNROLL, d_blk, (zero, zero))
            outs[b][pl.ds(g * 16, 16)] = acc0 + acc1
        pltpu.async_copy(
            outs[b], out_hbm.at[pl.ds(wbase + ci * C, C)], sems[3 * b + 2])

    for b in range(NBUF):
        fire(b, b)

    def loop_body(i, carry):
        for b in range(NBUF):
            ci = i * NBUF + b

            @pl.when(ci < NCHUNK)
            def _():
                wait(b)

                @pl.when(ci >= NBUF)
                def _():
                    wait_out(b)

                compute(ci, b)

                @pl.when(ci + NBUF < NCHUNK)
                def _():
                    fire(ci + NBUF, b)

        return carry

    lax.fori_loop(0, (NCHUNK + NBUF - 1) // NBUF, loop_body, 0)
    for b in range(NBUF):
        wait_out(b)


def kernel(node_src_feats, node_tgt_feats, edge_ids):
    eids = edge_ids.astype(jnp.int32)
    sids = eids[0]
    tids = eids[1]
    nn = node_src_feats.shape[0]
    src_w = lax.bitcast_convert_type(
        node_src_feats.astype(jnp.bfloat16).reshape(nn, W, 2), jnp.int32)
    tgt_w = lax.bitcast_convert_type(
        node_tgt_feats.astype(jnp.bfloat16).reshape(nn, W, 2), jnp.int32)
    mesh = plsc.VectorSubcoreMesh(core_axis_name="c", subcore_axis_name="s")
    fn = pl.kernel(
        _edge_dot_body,
        out_type=jax.ShapeDtypeStruct((E,), jnp.float32),
        mesh=mesh,
        scratch_types=[
            pltpu.VMEM((EPW,), jnp.int32),
            pltpu.VMEM((EPW,), jnp.int32),
        ] + [pltpu.VMEM((C, W), jnp.int32) for _ in range(2 * NBUF)]
          + [pltpu.VMEM((C,), jnp.float32) for _ in range(NBUF)]
          + [pltpu.SemaphoreType.DMA for _ in range(3 * NBUF)],
        compiler_params=pltpu.CompilerParams(
            needs_layout_passes=False, use_tc_tiling_on_sc=False),
    )
    return fn(src_w, tgt_w, sids, tids)


# speedup vs baseline: 10.2457x; 1.1270x over previous
import jax
import jax.numpy as jnp
from jax import lax
from jax.experimental import pallas as pl
from jax.experimental.pallas import tpu as pltpu
from jax.experimental.pallas import tpu_sc as plsc

D = 128
E = 320000
NC = 2
NS = 16
NW = NC * NS
EPW = E // NW
C = 80
NCHUNK = EPW // C
NBUF = 4
UNROLL = 8
W = D // 2


def _edge_dot_body(src_hbm, tgt_hbm, sid_hbm, tid_hbm, out_hbm,
                   sidx_v, tidx_v, out_v, *bufs_and_sems):
    srows = bufs_and_sems[0:NBUF]
    trows = bufs_and_sems[NBUF:2 * NBUF]
    sems = bufs_and_sems[2 * NBUF:]
    wid = lax.axis_index("s") * NC + lax.axis_index("c")
    wbase = wid * EPW

    pltpu.sync_copy(sid_hbm.at[pl.ds(wbase, EPW)], sidx_v)
    pltpu.sync_copy(tid_hbm.at[pl.ds(wbase, EPW)], tidx_v)

    def fire(ci, b):
        pltpu.async_copy(
            src_hbm.at[sidx_v.at[pl.ds(ci * C, C)]], srows[b], sems[2 * b])
        pltpu.async_copy(
            tgt_hbm.at[tidx_v.at[pl.ds(ci * C, C)]], trows[b], sems[2 * b + 1])

    def wait(b):
        pltpu.make_async_copy(
            src_hbm.at[pl.ds(0, C)], srows[b], sems[2 * b]).wait()
        pltpu.make_async_copy(
            tgt_hbm.at[pl.ds(0, C)], trows[b], sems[2 * b + 1]).wait()

    def compute(ci, b):
        sb = srows[b]
        tb = trows[b]
        lane = lax.iota(jnp.int32, 16)
        for g in range(C // 16):
            rows = lane + g * 16
            zero = jnp.zeros((16,), jnp.float32)

            def d_blk(k, carry):
                acc0, acc1 = carry
                base = k * UNROLL
                for j in range(UNROLL):
                    col = (jnp.full((16,), base + j, jnp.int32) + lane) & (W - 1)
                    s = plsc.load_gather(sb, [rows, col])
                    t = plsc.load_gather(tb, [rows, col])
                    p = plsc.bitcast(s, jnp.bfloat16) * plsc.bitcast(t, jnp.bfloat16)
                    pe, po = plsc.unpack(p, format=plsc.PackFormat.INTERLEAVED)
                    acc0 = acc0 + pe
                    acc1 = acc1 + po
                return acc0, acc1

            acc0, acc1 = lax.fori_loop(0, W // UNROLL, d_blk, (zero, zero))
            out_v[pl.ds(ci * C + g * 16, 16)] = acc0 + acc1

    for b in range(NBUF):
        fire(b, b)

    def loop_body(i, carry):
        for b in range(NBUF):
            ci = i * NBUF + b

            @pl.when(ci < NCHUNK)
            def _():
                wait(b)
                compute(ci, b)

                @pl.when(ci + NBUF < NCHUNK)
                def _():
                    fire(ci + NBUF, b)

        return carry

    lax.fori_loop(0, (NCHUNK + NBUF - 1) // NBUF, loop_body, 0)
    pltpu.sync_copy(out_v, out_hbm.at[pl.ds(wbase, EPW)])


def kernel(node_src_feats, node_tgt_feats, edge_ids):
    eids = edge_ids.astype(jnp.int32)
    sids = eids[0]
    tids = eids[1]
    nn = node_src_feats.shape[0]
    src_w = lax.bitcast_convert_type(
        node_src_feats.astype(jnp.bfloat16).reshape(nn, W, 2), jnp.int32)
    tgt_w = lax.bitcast_convert_type(
        node_tgt_feats.astype(jnp.bfloat16).reshape(nn, W, 2), jnp.int32)
    mesh = plsc.VectorSubcoreMesh(core_axis_name="c", subcore_axis_name="s")
    fn = pl.kernel(
        _edge_dot_body,
        out_type=jax.ShapeDtypeStruct((E,), jnp.float32),
        mesh=mesh,
        scratch_types=[
            pltpu.VMEM((EPW,), jnp.int32),
            pltpu.VMEM((EPW,), jnp.int32),
            pltpu.VMEM((EPW,), jnp.float32),
        ] + [pltpu.VMEM((C, W), jnp.int32) for _ in range(2 * NBUF)]
          + [pltpu.SemaphoreType.DMA for _ in range(2 * NBUF)],
        compiler_params=pltpu.CompilerParams(
            needs_layout_passes=False, use_tc_tiling_on_sc=False),
    )
    return fn(src_w, tgt_w, sids, tids)


# speedup vs baseline: 12.0915x; 1.1801x over previous
import jax
import jax.numpy as jnp
from jax import lax
from jax.experimental import pallas as pl
from jax.experimental.pallas import tpu as pltpu
from jax.experimental.pallas import tpu_sc as plsc

D = 128
E = 320000
NC = 2
NS = 16
NW = NC * NS
EPW = E // NW
C = 80
NCHUNK = EPW // C
NBUF = 4
UNROLL = 8
W = D // 2


def _edge_dot_body(src_hbm, tgt_hbm, sid_hbm, tid_hbm, out_hbm,
                   sidx_v, tidx_v, out_v, *bufs_and_sems):
    srows = bufs_and_sems[0:NBUF]
    trows = bufs_and_sems[NBUF:2 * NBUF]
    sems = bufs_and_sems[2 * NBUF:]
    wid = lax.axis_index("s") * NC + lax.axis_index("c")
    wbase = wid * EPW

    pltpu.sync_copy(sid_hbm.at[pl.ds(wbase, EPW)], sidx_v)
    pltpu.sync_copy(tid_hbm.at[pl.ds(wbase, EPW)], tidx_v)

    def fire(ci, b):
        pltpu.async_copy(
            src_hbm.at[sidx_v.at[pl.ds(ci * C, C)]], srows[b], sems[2 * b])
        pltpu.async_copy(
            tgt_hbm.at[tidx_v.at[pl.ds(ci * C, C)]], trows[b], sems[2 * b + 1])

    def wait(b):
        pltpu.make_async_copy(
            src_hbm.at[pl.ds(0, C)], srows[b], sems[2 * b]).wait()
        pltpu.make_async_copy(
            tgt_hbm.at[pl.ds(0, C)], trows[b], sems[2 * b + 1]).wait()

    def compute(ci, b):
        sb = srows[b]
        tb = trows[b]
        lane = lax.iota(jnp.int32, 16)
        for g in range(C // 16):
            rows = lane + g * 16
            zero = jnp.zeros((16,), jnp.float32)

            def d_blk(k, carry):
                acc0, acc1 = carry
                base = k * UNROLL
                for j in range(UNROLL):
                    col = (jnp.full((16,), base + j, jnp.int32) + lane) & (D - 1)
                    s = plsc.load_gather(sb, [rows, col])
                    t = plsc.load_gather(tb, [rows, col])
                    if j % 2 == 0:
                        acc0 = acc0 + s * t
                    else:
                        acc1 = acc1 + s * t
                return acc0, acc1

            acc0, acc1 = lax.fori_loop(0, D // UNROLL, d_blk, (zero, zero))
            out_v[pl.ds(ci * C + g * 16, 16)] = acc0 + acc1

    for b in range(NBUF):
        fire(b, b)

    def loop_body(i, carry):
        for b in range(NBUF):
            ci = i * NBUF + b

            @pl.when(ci < NCHUNK)
            def _():
                wait(b)
                compute(ci, b)

                @pl.when(ci + NBUF < NCHUNK)
                def _():
                    fire(ci + NBUF, b)

        return carry

    lax.fori_loop(0, (NCHUNK + NBUF - 1) // NBUF, loop_body, 0)
    pltpu.sync_copy(out_v, out_hbm.at[pl.ds(wbase, EPW)])


def kernel(node_src_feats, node_tgt_feats, edge_ids):
    eids = edge_ids.astype(jnp.int32)
    sids = eids[0]
    tids = eids[1]
    mesh = plsc.VectorSubcoreMesh(core_axis_name="c", subcore_axis_name="s")
    fn = pl.kernel(
        _edge_dot_body,
        out_type=jax.ShapeDtypeStruct((E,), jnp.float32),
        mesh=mesh,
        scratch_types=[
            pltpu.VMEM((EPW,), jnp.int32),
            pltpu.VMEM((EPW,), jnp.int32),
            pltpu.VMEM((EPW,), jnp.float32),
        ] + [pltpu.VMEM((C, D), jnp.float32) for _ in range(2 * NBUF)]
          + [pltpu.SemaphoreType.DMA for _ in range(2 * NBUF)],
        compiler_params=pltpu.CompilerParams(
            needs_layout_passes=False, use_tc_tiling_on_sc=False),
    )
    return fn(node_src_feats, node_tgt_feats, sids, tids)
